# baseline (speedup 1.0000x reference)
import functools

import jax
import jax.numpy as jnp
from jax import lax
from jax.experimental import pallas as pl
from jax.experimental.pallas import tpu as pltpu
from jax.experimental.pallas import tpu_sc as plsc

_V = 8192
_MAX_N = 4
_SEQ = 128
_SMOOTH = 1e-10
_L = 16
_RB = 16
_NB = _SEQ // _RB


def _sc_ref_hist(ids):
  num_refs, seq = ids.shape
  mesh = plsc.VectorSubcoreMesh(core_axis_name="c", subcore_axis_name="s",
                                num_cores=1)

  @functools.partial(
      pl.kernel,
      out_type=jax.ShapeDtypeStruct((num_refs, _V), jnp.float32),
      mesh=mesh,
      compiler_params=pltpu.CompilerParams(needs_layout_passes=False),
      scratch_types=[
          pltpu.VMEM((seq,), jnp.int32),
          pltpu.VMEM((_V,), jnp.float32),
      ],
  )
  def k(ids_hbm, out_hbm, ids_v, cnt_v):
    wid = lax.axis_index("s")

    @pl.when(wid < num_refs)
    def _():
      r = wid
      pltpu.sync_copy(ids_hbm.at[r], ids_v)

      def zero_body(i, c):
        z = jnp.zeros((_L,), jnp.float32)
        for u in range(4):
          cnt_v[pl.ds(i * 4 * _L + u * _L, _L)] = z
        return c

      lax.fori_loop(0, _V // (4 * _L), zero_body, 0)

      lane = lax.broadcasted_iota(jnp.int32, (_L,), 0)
      ones = jnp.ones((_L,), jnp.float32)
      for g in range(seq // _L):
        idx = ids_v[pl.ds(g * _L, _L)]
        for l in range(_L):
          plsc.addupdate_scatter(cnt_v, [idx], ones, mask=(lane == l))

      pltpu.sync_copy(cnt_v, out_hbm.at[r])

  return k(ids)


def _tc1_body(x_ref, out_ref):
  i = pl.program_id(0)
  x = x_ref[...]
  m = jnp.max(x, axis=1, keepdims=True)
  e = jnp.exp(x - m)
  s = jnp.sum(e, axis=1, keepdims=True)
  d = e / s

  tpart = jnp.sum(d, axis=0, keepdims=True)

  @pl.when(i == 0)
  def _():
    out_ref[...] = jnp.zeros((2 * _MAX_N, _V), jnp.float32)
    acc = jnp.zeros((1, _V), jnp.float32)
    for k in range(1, _MAX_N):
      acc = acc + d[k - 1:k, :]
      out_ref[k:k + 1, :] = acc

  out_ref[0:1, :] += tpart

  @pl.when(i == _NB - 1)
  def _():
    acc = jnp.zeros((1, _V), jnp.float32)
    for k in range(1, _MAX_N):
      acc = acc + d[_RB - k:_RB - k + 1, :]
      out_ref[_MAX_N - 1 + k:_MAX_N + k, :] = acc


def _tc1_colsums(x):
  return pl.pallas_call(
      _tc1_body,
      grid=(_NB,),
      in_specs=[pl.BlockSpec((_RB, _V), lambda i: (i, 0))],
      out_specs=pl.BlockSpec((2 * _MAX_N, _V), lambda i: (0, 0)),
      out_shape=jax.ShapeDtypeStruct((2 * _MAX_N, _V), jnp.float32),
  )(x)


def _tc2_body(v_ref, fc_ref, ids_ref, out_ref):
  t = v_ref[0:1, :]
  zero = jnp.zeros((1, _V), jnp.float32)
  first = [zero] + [v_ref[k:k + 1, :] for k in range(1, _MAX_N)]
  last = [zero] + [v_ref[_MAX_N - 1 + k:_MAX_N + k, :]
                   for k in range(1, _MAX_N)]

  num_refs = fc_ref.shape[0]
  iota = lax.broadcasted_iota(jnp.int32, (1, _V), 1)
  g = []
  h = []
  for r in range(num_refs):
    gr = [zero]
    hr = [zero]
    for k in range(1, _MAX_N):
      gr.append(gr[-1] + (iota == ids_ref[r, k - 1]).astype(jnp.float32))
      hr.append(hr[-1] + (iota == ids_ref[r, _SEQ - k]).astype(jnp.float32))
    g.append(gr)
    h.append(hr)

  logp = jnp.float32(0.0)
  for n in range(1, _MAX_N + 1):
    cacc = zero
    macc = zero
    for j in range(n):
      c = t - first[j] - last[n - 1 - j]
      rmax = None
      for r in range(num_refs):
        rc = fc_ref[r:r + 1, :] - g[r][j] - h[r][n - 1 - j]
        rmax = rc if rmax is None else jnp.maximum(rmax, rc)
      cacc = cacc + c
      macc = macc + jnp.minimum(c, rmax)
    tclip = jnp.sum(macc)
    tcand = jnp.sum(cacc)
    prec = (tclip + _SMOOTH) / (tcand + _SMOOTH)
    logp = logp + jnp.log(jnp.maximum(prec, _SMOOTH))
  out_ref[...] = jnp.exp(logp / _MAX_N)[None, None]


def _tc2_combine(v, fc, ids):
  return pl.pallas_call(
      _tc2_body,
      in_specs=[
          pl.BlockSpec(memory_space=pltpu.VMEM),
          pl.BlockSpec(memory_space=pltpu.VMEM),
          pl.BlockSpec(memory_space=pltpu.SMEM),
      ],
      out_shape=jax.ShapeDtypeStruct((1, 1), jnp.float32),
  )(v, fc, ids)


@jax.jit
def kernel(candidate_input, reference_ids_list):
  fc = _sc_ref_hist(reference_ids_list)
  v = _tc1_colsums(candidate_input)
  return _tc2_combine(v, fc, reference_ids_list)[0, 0]


# speedup vs baseline: 1.0776x; 1.0776x over previous
import functools

import jax
import jax.numpy as jnp
from jax import lax
from jax.experimental import pallas as pl
from jax.experimental.pallas import tpu as pltpu
from jax.experimental.pallas import tpu_sc as plsc

_V = 8192
_MAX_N = 4
_SEQ = 128
_SMOOTH = 1e-10
_L = 16
_RB = 16
_NB = _SEQ // _RB


def _sc_ref_hist(ids):
  num_refs, seq = ids.shape
  mesh = plsc.VectorSubcoreMesh(core_axis_name="c", subcore_axis_name="s",
                                num_cores=1)

  @functools.partial(
      pl.kernel,
      out_type=jax.ShapeDtypeStruct((num_refs, _V), jnp.float32),
      mesh=mesh,
      compiler_params=pltpu.CompilerParams(needs_layout_passes=False,
                                           skip_device_barrier=True),
      scratch_types=[
          pltpu.VMEM((seq,), jnp.int32),
          pltpu.VMEM((_V,), jnp.float32),
      ],
  )
  def k(ids_hbm, out_hbm, ids_v, cnt_v):
    wid = lax.axis_index("s")

    @pl.when(wid < num_refs)
    def _():
      r = wid
      pltpu.sync_copy(ids_hbm.at[r], ids_v)

      def zero_body(i, c):
        z = jnp.zeros((_L,), jnp.float32)
        for u in range(4):
          cnt_v[pl.ds(i * 4 * _L + u * _L, _L)] = z
        return c

      lax.fori_loop(0, _V // (4 * _L), zero_body, 0)

      lane = lax.broadcasted_iota(jnp.int32, (_L,), 0)
      ones = jnp.ones((_L,), jnp.float32)
      for g in range(seq // _L):
        idx = ids_v[pl.ds(g * _L, _L)]
        for l in range(_L):
          plsc.addupdate_scatter(cnt_v, [idx], ones, mask=(lane == l))

      pltpu.sync_copy(cnt_v, out_hbm.at[r])

  return k(ids)


def _tc1_body(x_ref, out_ref):
  x = x_ref[...]
  m = jnp.max(x, axis=1, keepdims=True)
  e = jnp.exp(x - m)
  s = jnp.sum(e, axis=1, keepdims=True)
  d = e / s

  rows = [jnp.sum(d, axis=0, keepdims=True)]
  acc = jnp.zeros((1, _V), jnp.float32)
  for k in range(1, _MAX_N):
    acc = acc + d[k - 1:k, :]
    rows.append(acc)
  acc = jnp.zeros((1, _V), jnp.float32)
  for k in range(1, _MAX_N):
    acc = acc + d[_SEQ - k:_SEQ - k + 1, :]
    rows.append(acc)
  rows.append(jnp.zeros((1, _V), jnp.float32))
  out_ref[...] = jnp.concatenate(rows, axis=0)


def _tc1_colsums(x):
  return pl.pallas_call(
      _tc1_body,
      out_shape=jax.ShapeDtypeStruct((2 * _MAX_N, _V), jnp.float32),
  )(x)


def _tc2_body(v_ref, fc_ref, ids_ref, out_ref):
  t = v_ref[0:1, :]
  zero = jnp.zeros((1, _V), jnp.float32)
  first = [zero] + [v_ref[k:k + 1, :] for k in range(1, _MAX_N)]
  last = [zero] + [v_ref[_MAX_N - 1 + k:_MAX_N + k, :]
                   for k in range(1, _MAX_N)]

  num_refs = fc_ref.shape[0]
  iota = lax.broadcasted_iota(jnp.int32, (1, _V), 1)
  g = []
  h = []
  for r in range(num_refs):
    gr = [zero]
    hr = [zero]
    for k in range(1, _MAX_N):
      gr.append(gr[-1] + (iota == ids_ref[r, k - 1]).astype(jnp.float32))
      hr.append(hr[-1] + (iota == ids_ref[r, _SEQ - k]).astype(jnp.float32))
    g.append(gr)
    h.append(hr)

  logp = jnp.float32(0.0)
  for n in range(1, _MAX_N + 1):
    cacc = zero
    macc = zero
    for j in range(n):
      c = t - first[j] - last[n - 1 - j]
      rmax = None
      for r in range(num_refs):
        rc = fc_ref[r:r + 1, :] - g[r][j] - h[r][n - 1 - j]
        rmax = rc if rmax is None else jnp.maximum(rmax, rc)
      cacc = cacc + c
      macc = macc + jnp.minimum(c, rmax)
    tclip = jnp.sum(macc)
    tcand = jnp.sum(cacc)
    prec = (tclip + _SMOOTH) / (tcand + _SMOOTH)
    logp = logp + jnp.log(jnp.maximum(prec, _SMOOTH))
  out_ref[...] = jnp.exp(logp / _MAX_N)[None, None]


def _tc2_combine(v, fc, ids):
  return pl.pallas_call(
      _tc2_body,
      in_specs=[
          pl.BlockSpec(memory_space=pltpu.VMEM),
          pl.BlockSpec(memory_space=pltpu.VMEM),
          pl.BlockSpec(memory_space=pltpu.SMEM),
      ],
      out_shape=jax.ShapeDtypeStruct((1, 1), jnp.float32),
  )(v, fc, ids)


@jax.jit
def kernel(candidate_input, reference_ids_list):
  fc = _sc_ref_hist(reference_ids_list)
  v = _tc1_colsums(candidate_input)
  return _tc2_combine(v, fc, reference_ids_list)[0, 0]


# speedup vs baseline: 1.0778x; 1.0002x over previous
import functools

import jax
import jax.numpy as jnp
from jax import lax
from jax.experimental import pallas as pl
from jax.experimental.pallas import tpu as pltpu
from jax.experimental.pallas import tpu_sc as plsc

_V = 8192
_MAX_N = 4
_SEQ = 128
_SMOOTH = 1e-10
_L = 16
_RB = 16
_NB = _SEQ // _RB


def _sc_ref_hist(ids):
  num_refs, seq = ids.shape
  mesh = plsc.VectorSubcoreMesh(core_axis_name="c", subcore_axis_name="s",
                                num_cores=1)

  @functools.partial(
      pl.kernel,
      out_type=jax.ShapeDtypeStruct((num_refs, _V), jnp.float32),
      mesh=mesh,
      compiler_params=pltpu.CompilerParams(needs_layout_passes=False,
                                           skip_device_barrier=True,
                                           disable_bounds_checks=True,
                                           disable_semaphore_checks=True),
      scratch_types=[
          pltpu.VMEM((seq,), jnp.int32),
          pltpu.VMEM((_V,), jnp.float32),
      ],
  )
  def k(ids_hbm, out_hbm, ids_v, cnt_v):
    wid = lax.axis_index("s")

    @pl.when(wid < num_refs)
    def _():
      r = wid
      pltpu.sync_copy(ids_hbm.at[r], ids_v)

      def zero_body(i, c):
        z = jnp.zeros((_L,), jnp.float32)
        for u in range(4):
          cnt_v[pl.ds(i * 4 * _L + u * _L, _L)] = z
        return c

      lax.fori_loop(0, _V // (4 * _L), zero_body, 0)

      lane = lax.broadcasted_iota(jnp.int32, (_L,), 0)
      ones = jnp.ones((_L,), jnp.float32)
      for g in range(seq // _L):
        idx = ids_v[pl.ds(g * _L, _L)]
        for l in range(_L):
          plsc.addupdate_scatter(cnt_v, [idx], ones, mask=(lane == l))

      pltpu.sync_copy(cnt_v, out_hbm.at[r])

  return k(ids)


def _tc1_body(x_ref, out_ref):
  x = x_ref[...]
  m = jnp.max(x, axis=1, keepdims=True)
  e = jnp.exp(x - m)
  s = jnp.sum(e, axis=1, keepdims=True)
  d = e / s

  rows = [jnp.sum(d, axis=0, keepdims=True)]
  acc = jnp.zeros((1, _V), jnp.float32)
  for k in range(1, _MAX_N):
    acc = acc + d[k - 1:k, :]
    rows.append(acc)
  acc = jnp.zeros((1, _V), jnp.float32)
  for k in range(1, _MAX_N):
    acc = acc + d[_SEQ - k:_SEQ - k + 1, :]
    rows.append(acc)
  rows.append(jnp.zeros((1, _V), jnp.float32))
  out_ref[...] = jnp.concatenate(rows, axis=0)


def _tc1_colsums(x):
  return pl.pallas_call(
      _tc1_body,
      compiler_params=pltpu.CompilerParams(skip_device_barrier=True),
      out_shape=jax.ShapeDtypeStruct((2 * _MAX_N, _V), jnp.float32),
  )(x)


def _tc2_body(v_ref, fc_ref, ids_ref, out_ref):
  t = v_ref[0:1, :]
  zero = jnp.zeros((1, _V), jnp.float32)
  first = [zero] + [v_ref[k:k + 1, :] for k in range(1, _MAX_N)]
  last = [zero] + [v_ref[_MAX_N - 1 + k:_MAX_N + k, :]
                   for k in range(1, _MAX_N)]

  num_refs = fc_ref.shape[0]
  iota = lax.broadcasted_iota(jnp.int32, (1, _V), 1)
  g = []
  h = []
  for r in range(num_refs):
    gr = [zero]
    hr = [zero]
    for k in range(1, _MAX_N):
      gr.append(gr[-1] + (iota == ids_ref[r, k - 1]).astype(jnp.float32))
      hr.append(hr[-1] + (iota == ids_ref[r, _SEQ - k]).astype(jnp.float32))
    g.append(gr)
    h.append(hr)

  logp = jnp.float32(0.0)
  for n in range(1, _MAX_N + 1):
    cacc = zero
    macc = zero
    for j in range(n):
      c = t - first[j] - last[n - 1 - j]
      rmax = None
      for r in range(num_refs):
        rc = fc_ref[r:r + 1, :] - g[r][j] - h[r][n - 1 - j]
        rmax = rc if rmax is None else jnp.maximum(rmax, rc)
      cacc = cacc + c
      macc = macc + jnp.minimum(c, rmax)
    tclip = jnp.sum(macc)
    tcand = jnp.sum(cacc)
    prec = (tclip + _SMOOTH) / (tcand + _SMOOTH)
    logp = logp + jnp.log(jnp.maximum(prec, _SMOOTH))
  out_ref[...] = jnp.exp(logp / _MAX_N)[None, None]


def _tc2_combine(v, fc, ids):
  return pl.pallas_call(
      _tc2_body,
      in_specs=[
          pl.BlockSpec(memory_space=pltpu.VMEM),
          pl.BlockSpec(memory_space=pltpu.VMEM),
          pl.BlockSpec(memory_space=pltpu.SMEM),
      ],
      compiler_params=pltpu.CompilerParams(skip_device_barrier=True),
      out_shape=jax.ShapeDtypeStruct((1, 1), jnp.float32),
  )(v, fc, ids)


@jax.jit
def kernel(candidate_input, reference_ids_list):
  fc = _sc_ref_hist(reference_ids_list)
  v = _tc1_colsums(candidate_input)
  return _tc2_combine(v, fc, reference_ids_list)[0, 0]
